# baseline (speedup 1.0000x reference)
import jax
import jax.numpy as jnp
from jax import lax
from jax.experimental import pallas as pl
from jax.experimental.pallas import tpu as pltpu
from jax.experimental.pallas import tpu_sc as plsc

N, C, B = 320000, 128, 16
NC, NS, L = 2, 16, 16
NW = NC * NS
CV = C // L
EPS = 1e-5
SD = B * 2 * C

RB = 512
N_SC = 153600
N_TC = N - N_SC
PER_S = N_SC // NW
RS = 200
NCHS = PER_S // RS
PER_A = N // NW
RA = 200
NCHA = PER_A // RA


def _worker_id():
    return lax.axis_index("s") * NC + lax.axis_index("c")


def _local_starts(ids_hbm, ids_v, cnt_v, starts_v, lo, per):
    pltpu.sync_copy(ids_hbm.at[pl.ds(lo, per)], ids_v)
    cnt_v[...] = jnp.zeros((B,), jnp.int32)
    ones = jnp.ones((L,), jnp.int32)

    def _hist(i, carry):
        v = ids_v[pl.ds(i * L, L)]
        plsc.addupdate_scatter(cnt_v, [v], ones)
        return carry

    lax.fori_loop(0, per // L, _hist, 0)
    cnts = cnt_v[...]
    st16 = plsc.cumsum(cnts) - cnts
    starts_v[pl.ds(0, L)] = st16
    starts_v[pl.ds(L, L)] = jnp.full((L,), per, jnp.int32)
    return st16


def _seg_bounds(starts_v, b, g_lo, g_hi):
    sv = starts_v[pl.ds(b, L)]
    a = jnp.maximum(sv[0], g_lo)
    e = jnp.minimum(sv[1], g_hi)
    return a, e


def _stats_body(data_hbm, ids_hbm, part_hbm, cnt_hbm,
                ids_v, acc_v, starts_v, cnt_v, buf0, buf1, sin0, sin1):
    wid = _worker_id()
    lo = wid * PER_S

    pltpu.async_copy(data_hbm.at[pl.ds(lo, RS), :], buf0, sin0)
    pltpu.async_copy(data_hbm.at[pl.ds(lo + RS, RS), :], buf1, sin1)

    st16 = _local_starts(ids_hbm, ids_v, cnt_v, starts_v, lo, PER_S)

    z = jnp.zeros((L,), jnp.float32)

    def _zero(i, carry):
        acc_v[pl.ds(i * L, L)] = z
        return carry

    lax.fori_loop(0, SD // L, _zero, 0)

    def _chunk(g, buf):
        g_lo = g * RS
        b_first = jnp.sum((st16 <= g_lo).astype(jnp.int32)) - 1
        b_last = jnp.sum((st16 <= g_lo + RS - 1).astype(jnp.int32)) - 1

        def _seg(b, carry):
            a, e = _seg_bounds(starts_v, b, g_lo, g_lo + RS)

            @pl.when(e > a)
            def _():
                def _rows(r, rcarry):
                    rr = r - g_lo
                    x = [buf[rr, pl.ds(j * L, L)] for j in range(CV)]
                    return (tuple(rcarry[j] + x[j] for j in range(CV))
                            + tuple(rcarry[CV + j] + x[j] * x[j]
                                    for j in range(CV)))

                init = tuple(jnp.zeros((L,), jnp.float32)
                             for _ in range(2 * CV))
                acc = lax.fori_loop(a, e, _rows, init)
                for j in range(CV):
                    plsc.addupdate(
                        acc_v.at[pl.ds(b * 2 * C + j * L, L)], acc[j])
                    plsc.addupdate(
                        acc_v.at[pl.ds(b * 2 * C + C + j * L, L)],
                        acc[CV + j])

            return carry

        lax.fori_loop(b_first, b_last + 1, _seg, 0)

    def _pair(p, carry):
        g0 = 2 * p
        pltpu.make_async_copy(
            data_hbm.at[pl.ds(lo + g0 * RS, RS), :], buf0, sin0).wait()
        _chunk(g0, buf0)

        @pl.when(g0 + 2 < NCHS)
        def _():
            pltpu.async_copy(
                data_hbm.at[pl.ds(lo + (g0 + 2) * RS, RS), :], buf0, sin0)

        pltpu.make_async_copy(
            data_hbm.at[pl.ds(lo + (g0 + 1) * RS, RS), :], buf1, sin1).wait()
        _chunk(g0 + 1, buf1)

        @pl.when(g0 + 3 < NCHS)
        def _():
            pltpu.async_copy(
                data_hbm.at[pl.ds(lo + (g0 + 3) * RS, RS), :], buf1, sin1)

        return carry

    lax.fori_loop(0, NCHS // 2, _pair, 0)

    pltpu.sync_copy(acc_v, part_hbm.at[wid])
    pltpu.sync_copy(cnt_v, cnt_hbm.at[wid])


def _tc_stats_body(data_ref, ids_ref, out_ref):
    i = pl.program_id(0)
    x = data_ref[...]
    idsb = ids_ref[0, 0, :]
    iota = lax.broadcasted_iota(jnp.int32, (B, RB), 0)
    onehot_t = (iota == idsb[None, :]).astype(jnp.float32)
    s = jnp.dot(onehot_t, x, preferred_element_type=jnp.float32)
    q = jnp.dot(onehot_t, x * x, preferred_element_type=jnp.float32)
    cnt = jnp.sum(onehot_t, axis=1, keepdims=True)
    blk = jnp.concatenate(
        [s, q, jnp.broadcast_to(cnt, (B, C))], axis=1)

    @pl.when(i == 0)
    def _():
        out_ref[...] = jnp.zeros_like(out_ref)

    out_ref[...] += blk


def _tc_finalize_body(part_ref, cnt_ref, ptc_ref, w_ref, bias_ref, out_ref):
    stats = jnp.sum(part_ref[...], axis=0)
    ptc = ptc_ref[...]
    s_sum = stats[:, :C] + ptc[:, :C]
    q_sum = stats[:, C:] + ptc[:, C:2 * C]
    n_sc = jnp.sum(cnt_ref[...].astype(jnp.float32), axis=0)
    n = n_sc[:, None] + ptc[:, 2 * C:2 * C + 1]
    eta = 1.0 / (n + EPS)
    mean = s_sum * eta
    var = eta * q_sum - (eta * eta) * (s_sum * s_sum) * (2.0 - n * eta)
    inv = lax.rsqrt(var + EPS)
    scale = inv * w_ref[...]
    shift = bias_ref[...] - mean * scale
    out_ref[0, :, :] = scale
    out_ref[1, :, :] = shift


def _apply_body(data_hbm, ids_hbm, scsh_hbm, out_hbm,
                ids_v, starts_v, cnt_v, sc_v, sh_v,
                bin0, bin1, bout0, bout1,
                sin0, sin1, sout0, sout1):
    wid = _worker_id()
    lo = wid * PER_A

    pltpu.async_copy(data_hbm.at[pl.ds(lo, RA), :], bin0, sin0)
    pltpu.async_copy(data_hbm.at[pl.ds(lo + RA, RA), :], bin1, sin1)

    st16 = _local_starts(ids_hbm, ids_v, cnt_v, starts_v, lo, PER_A)

    pltpu.sync_copy(scsh_hbm.at[0], sc_v)
    pltpu.sync_copy(scsh_hbm.at[1], sh_v)

    def _chunk(g, bin_, bout):
        g_lo = g * RA
        b_first = jnp.sum((st16 <= g_lo).astype(jnp.int32)) - 1
        b_last = jnp.sum((st16 <= g_lo + RA - 1).astype(jnp.int32)) - 1

        def _seg(b, carry):
            a, e = _seg_bounds(starts_v, b, g_lo, g_lo + RA)

            @pl.when(e > a)
            def _():
                sc = [sc_v[pl.ds(b * C + j * L, L)] for j in range(CV)]
                sh = [sh_v[pl.ds(b * C + j * L, L)] for j in range(CV)]

                @plsc.parallel_loop(a - g_lo, e - g_lo)
                def _rows(rr):
                    x = [bin_[rr, pl.ds(j * L, L)] for j in range(CV)]
                    for j in range(CV):
                        bout[rr, pl.ds(j * L, L)] = x[j] * sc[j] + sh[j]

            return carry

        lax.fori_loop(b_first, b_last + 1, _seg, 0)

    def _pair(p, carry):
        g0 = 2 * p

        @pl.when(g0 >= 2)
        def _():
            pltpu.make_async_copy(
                bout0, out_hbm.at[pl.ds(lo + (g0 - 2) * RA, RA), :],
                sout0).wait()

        pltpu.make_async_copy(
            data_hbm.at[pl.ds(lo + g0 * RA, RA), :], bin0, sin0).wait()
        _chunk(g0, bin0, bout0)
        pltpu.async_copy(
            bout0, out_hbm.at[pl.ds(lo + g0 * RA, RA), :], sout0)

        @pl.when(g0 + 2 < NCHA)
        def _():
            pltpu.async_copy(
                data_hbm.at[pl.ds(lo + (g0 + 2) * RA, RA), :], bin0, sin0)

        @pl.when(g0 >= 2)
        def _():
            pltpu.make_async_copy(
                bout1, out_hbm.at[pl.ds(lo + (g0 - 1) * RA, RA), :],
                sout1).wait()

        pltpu.make_async_copy(
            data_hbm.at[pl.ds(lo + (g0 + 1) * RA, RA), :], bin1, sin1).wait()
        _chunk(g0 + 1, bin1, bout1)
        pltpu.async_copy(
            bout1, out_hbm.at[pl.ds(lo + (g0 + 1) * RA, RA), :], sout1)

        @pl.when(g0 + 3 < NCHA)
        def _():
            pltpu.async_copy(
                data_hbm.at[pl.ds(lo + (g0 + 3) * RA, RA), :], bin1, sin1)

        return carry

    lax.fori_loop(0, NCHA // 2, _pair, 0)

    pltpu.make_async_copy(
        bout0, out_hbm.at[pl.ds(lo + (NCHA - 2) * RA, RA), :], sout0).wait()
    pltpu.make_async_copy(
        bout1, out_hbm.at[pl.ds(lo + (NCHA - 1) * RA, RA), :], sout1).wait()


def kernel(data, batch_id, batch_size, weights, bias):
    del batch_size
    ids = batch_id.astype(jnp.int32)
    mesh = plsc.VectorSubcoreMesh(core_axis_name="c", subcore_axis_name="s",
                                  num_cores=NC, num_subcores=NS)

    sc_stats = pl.kernel(
        _stats_body,
        out_type=(jax.ShapeDtypeStruct((NW, SD), jnp.float32),
                  jax.ShapeDtypeStruct((NW, B), jnp.int32)),
        mesh=mesh,
        scratch_types=[
            pltpu.VMEM((PER_S,), jnp.int32),
            pltpu.VMEM((SD,), jnp.float32),
            pltpu.VMEM((2 * B,), jnp.int32),
            pltpu.VMEM((B,), jnp.int32),
            pltpu.VMEM((RS, C), jnp.float32),
            pltpu.VMEM((RS, C), jnp.float32),
            pltpu.SemaphoreType.DMA,
            pltpu.SemaphoreType.DMA,
        ],
        compiler_params=pltpu.CompilerParams(needs_layout_passes=False),
        name="octree_in_stats",
    )
    part, cnt = sc_stats(data, ids)

    ids_tc = ids[N_SC:].reshape(N_TC // RB, 1, RB)
    part_tc = pl.pallas_call(
        _tc_stats_body,
        grid=(N_TC // RB,),
        in_specs=[
            pl.BlockSpec((RB, C), lambda i: (N_SC // RB + i, 0)),
            pl.BlockSpec((1, 1, RB), lambda i: (i, 0, 0)),
        ],
        out_specs=pl.BlockSpec((B, 3 * C), lambda i: (0, 0)),
        out_shape=jax.ShapeDtypeStruct((B, 3 * C), jnp.float32),
        name="octree_in_tc_stats",
    )(data, ids_tc)

    scsh = pl.pallas_call(
        _tc_finalize_body,
        in_specs=[
            pl.BlockSpec((NW, B, 2 * C), lambda: (0, 0, 0)),
            pl.BlockSpec((NW, B), lambda: (0, 0)),
            pl.BlockSpec((B, 3 * C), lambda: (0, 0)),
            pl.BlockSpec((1, C), lambda: (0, 0)),
            pl.BlockSpec((1, C), lambda: (0, 0)),
        ],
        out_specs=pl.BlockSpec((2, B, C), lambda: (0, 0, 0)),
        out_shape=jax.ShapeDtypeStruct((2, B, C), jnp.float32),
        name="octree_in_tc_finalize",
    )(part.reshape(NW, B, 2 * C), cnt, part_tc, weights, bias)

    scsh2 = scsh.reshape(2, B * C)

    apply_k = pl.kernel(
        _apply_body,
        out_type=jax.ShapeDtypeStruct((N, C), jnp.float32),
        mesh=mesh,
        scratch_types=[
            pltpu.VMEM((PER_A,), jnp.int32),
            pltpu.VMEM((2 * B,), jnp.int32),
            pltpu.VMEM((B,), jnp.int32),
            pltpu.VMEM((B * C,), jnp.float32),
            pltpu.VMEM((B * C,), jnp.float32),
            pltpu.VMEM((RA, C), jnp.float32),
            pltpu.VMEM((RA, C), jnp.float32),
            pltpu.VMEM((RA, C), jnp.float32),
            pltpu.VMEM((RA, C), jnp.float32),
            pltpu.SemaphoreType.DMA,
            pltpu.SemaphoreType.DMA,
            pltpu.SemaphoreType.DMA,
            pltpu.SemaphoreType.DMA,
        ],
        compiler_params=pltpu.CompilerParams(needs_layout_passes=False),
        name="octree_in_apply",
    )
    return apply_k(data, ids, scsh2)


# speedup vs baseline: 1.5669x; 1.5669x over previous
import jax
import jax.numpy as jnp
from jax import lax
from jax.experimental import pallas as pl
from jax.experimental.pallas import tpu as pltpu
from jax.experimental.pallas import tpu_sc as plsc

N, C, B = 320000, 128, 16
NC, NS, L = 2, 16, 16
NW = NC * NS
PER = N // NW
RS = 200
NCHS = PER // RS
RA = 200
NCHA = PER // RA
CV = C // L
EPS = 1e-5
SD = B * 2 * C


def _worker_id():
    return lax.axis_index("s") * NC + lax.axis_index("c")


def _local_starts(ids_hbm, ids_v, cnt_v, starts_v, lo):
    pltpu.sync_copy(ids_hbm.at[pl.ds(lo, PER)], ids_v)
    cnt_v[...] = jnp.zeros((B,), jnp.int32)
    ones = jnp.ones((L,), jnp.int32)

    def _hist(i, carry):
        v = ids_v[pl.ds(i * L, L)]
        plsc.addupdate_scatter(cnt_v, [v], ones)
        return carry

    lax.fori_loop(0, PER // L, _hist, 0)
    cnts = cnt_v[...]
    st16 = plsc.cumsum(cnts) - cnts
    starts_v[pl.ds(0, L)] = st16
    starts_v[pl.ds(L, L)] = jnp.full((L,), PER, jnp.int32)
    return st16


def _seg_bounds(starts_v, b, g_lo, g_hi):
    sv = starts_v[pl.ds(b, L)]
    a = jnp.maximum(sv[0], g_lo)
    e = jnp.minimum(sv[1], g_hi)
    return a, e


def _stats_body(data_hbm, ids_hbm, part_hbm, cnt_hbm,
                ids_v, acc_v, starts_v, cnt_v, buf0, buf1, sin0, sin1):
    wid = _worker_id()
    lo = wid * PER

    pltpu.async_copy(data_hbm.at[pl.ds(lo, RS), :], buf0, sin0)
    pltpu.async_copy(data_hbm.at[pl.ds(lo + RS, RS), :], buf1, sin1)

    st16 = _local_starts(ids_hbm, ids_v, cnt_v, starts_v, lo)

    z = jnp.zeros((L,), jnp.float32)

    def _zero(i, carry):
        acc_v[pl.ds(i * L, L)] = z
        return carry

    lax.fori_loop(0, SD // L, _zero, 0)

    def _chunk(g, buf):
        g_lo = g * RS
        b_first = jnp.sum((st16 <= g_lo).astype(jnp.int32)) - 1
        b_last = jnp.sum((st16 <= g_lo + RS - 1).astype(jnp.int32)) - 1

        def _seg(b, carry):
            a, e = _seg_bounds(starts_v, b, g_lo, g_lo + RS)

            @pl.when(e > a)
            def _():
                def _rows(r, rcarry):
                    rr = r - g_lo
                    x = [buf[rr, pl.ds(j * L, L)] for j in range(CV)]
                    return (tuple(rcarry[j] + x[j] for j in range(CV))
                            + tuple(rcarry[CV + j] + x[j] * x[j]
                                    for j in range(CV)))

                init = tuple(jnp.zeros((L,), jnp.float32)
                             for _ in range(2 * CV))
                acc = lax.fori_loop(a, e, _rows, init)
                for j in range(CV):
                    plsc.addupdate(
                        acc_v.at[pl.ds(b * 2 * C + j * L, L)], acc[j])
                    plsc.addupdate(
                        acc_v.at[pl.ds(b * 2 * C + C + j * L, L)],
                        acc[CV + j])

            return carry

        lax.fori_loop(b_first, b_last + 1, _seg, 0)

    def _pair(p, carry):
        g0 = 2 * p
        pltpu.make_async_copy(
            data_hbm.at[pl.ds(lo + g0 * RS, RS), :], buf0, sin0).wait()
        _chunk(g0, buf0)

        @pl.when(g0 + 2 < NCHS)
        def _():
            pltpu.async_copy(
                data_hbm.at[pl.ds(lo + (g0 + 2) * RS, RS), :], buf0, sin0)

        pltpu.make_async_copy(
            data_hbm.at[pl.ds(lo + (g0 + 1) * RS, RS), :], buf1, sin1).wait()
        _chunk(g0 + 1, buf1)

        @pl.when(g0 + 3 < NCHS)
        def _():
            pltpu.async_copy(
                data_hbm.at[pl.ds(lo + (g0 + 3) * RS, RS), :], buf1, sin1)

        return carry

    lax.fori_loop(0, NCHS // 2, _pair, 0)

    pltpu.sync_copy(acc_v, part_hbm.at[wid])
    pltpu.sync_copy(cnt_v, cnt_hbm.at[wid])


def _apply_body(data_hbm, part_hbm, cnt_hbm, w_hbm, bias_hbm,
                out_hbm,
                starts_v, stat_v, pbuf, cbuf, w_v, bias_v,
                sc_v, sh_v, bin0, bin1, bout0, bout1,
                sin0, sin1, sout0, sout1):
    wid = _worker_id()
    lo = wid * PER

    pltpu.async_copy(data_hbm.at[pl.ds(lo, RA), :], bin0, sin0)
    pltpu.async_copy(data_hbm.at[pl.ds(lo + RA, RA), :], bin1, sin1)

    pltpu.sync_copy(cnt_hbm, cbuf)
    tot = jnp.zeros((B,), jnp.int32)
    for w2 in range(NW):
        tot = tot + cbuf[w2, :]
    own = cbuf[wid, :]
    st16 = plsc.cumsum(own) - own
    starts_v[pl.ds(0, L)] = st16
    starts_v[pl.ds(L, L)] = jnp.full((L,), PER, jnp.int32)

    pltpu.sync_copy(w_hbm, w_v)
    pltpu.sync_copy(bias_hbm, bias_v)

    totf = tot.astype(jnp.float32)
    stat_v[pl.ds(0, L)] = totf
    stat_v[pl.ds(L, L)] = jnp.full((L,), 1.0, jnp.float32)

    b_lo = jnp.sum((st16 <= 0).astype(jnp.int32)) - 1
    b_hi = jnp.sum((st16 <= PER - 1).astype(jnp.int32)) - 1

    def _mkseg(b, carry):
        pltpu.sync_copy(part_hbm.at[:, pl.ds(b * 2 * C, 2 * C)], pbuf)

        def _racc(w2, acc):
            return tuple(acc[j] + pbuf[w2, pl.ds(j * L, L)]
                         for j in range(2 * CV))

        red = lax.fori_loop(
            0, NW, _racc,
            tuple(jnp.zeros((L,), jnp.float32) for _ in range(2 * CV)))

        n = jnp.full((L,), stat_v[pl.ds(b, L)][0])
        eta = 1.0 / (n + EPS)
        for j in range(CV):
            s_sum = red[j]
            q_sum = red[CV + j]
            mean = s_sum * eta
            var = eta * q_sum - (eta * eta) * (s_sum * s_sum) * (2.0 - n * eta)
            v = var + EPS
            i = plsc.bitcast(v, jnp.int32)
            y = plsc.bitcast(jnp.int32(0x5F3759DF) - (i >> 1), jnp.float32)
            for _ in range(3):
                y = y * (1.5 - 0.5 * v * y * y)
            wv = w_v[0, pl.ds(j * L, L)]
            bv = bias_v[0, pl.ds(j * L, L)]
            scale = y * wv
            sc_v[pl.ds(b * C + j * L, L)] = scale
            sh_v[pl.ds(b * C + j * L, L)] = bv - mean * scale
        return carry

    lax.fori_loop(b_lo, b_hi + 1, _mkseg, 0)

    def _chunk(g, bin_, bout):
        g_lo = g * RA
        b_first = jnp.sum((st16 <= g_lo).astype(jnp.int32)) - 1
        b_last = jnp.sum((st16 <= g_lo + RA - 1).astype(jnp.int32)) - 1

        def _seg(b, carry):
            a, e = _seg_bounds(starts_v, b, g_lo, g_lo + RA)

            @pl.when(e > a)
            def _():
                sc = [sc_v[pl.ds(b * C + j * L, L)] for j in range(CV)]
                sh = [sh_v[pl.ds(b * C + j * L, L)] for j in range(CV)]

                @plsc.parallel_loop(a - g_lo, e - g_lo)
                def _rows(rr):
                    x = [bin_[rr, pl.ds(j * L, L)] for j in range(CV)]
                    for j in range(CV):
                        bout[rr, pl.ds(j * L, L)] = x[j] * sc[j] + sh[j]

            return carry

        lax.fori_loop(b_first, b_last + 1, _seg, 0)

    def _pair(p, carry):
        g0 = 2 * p

        @pl.when(g0 >= 2)
        def _():
            pltpu.make_async_copy(
                bout0, out_hbm.at[pl.ds(lo + (g0 - 2) * RA, RA), :],
                sout0).wait()

        pltpu.make_async_copy(
            data_hbm.at[pl.ds(lo + g0 * RA, RA), :], bin0, sin0).wait()
        _chunk(g0, bin0, bout0)
        pltpu.async_copy(
            bout0, out_hbm.at[pl.ds(lo + g0 * RA, RA), :], sout0)

        @pl.when(g0 + 2 < NCHA)
        def _():
            pltpu.async_copy(
                data_hbm.at[pl.ds(lo + (g0 + 2) * RA, RA), :], bin0, sin0)

        @pl.when(g0 >= 2)
        def _():
            pltpu.make_async_copy(
                bout1, out_hbm.at[pl.ds(lo + (g0 - 1) * RA, RA), :],
                sout1).wait()

        pltpu.make_async_copy(
            data_hbm.at[pl.ds(lo + (g0 + 1) * RA, RA), :], bin1, sin1).wait()
        _chunk(g0 + 1, bin1, bout1)
        pltpu.async_copy(
            bout1, out_hbm.at[pl.ds(lo + (g0 + 1) * RA, RA), :], sout1)

        @pl.when(g0 + 3 < NCHA)
        def _():
            pltpu.async_copy(
                data_hbm.at[pl.ds(lo + (g0 + 3) * RA, RA), :], bin1, sin1)

        return carry

    lax.fori_loop(0, NCHA // 2, _pair, 0)

    pltpu.make_async_copy(
        bout0, out_hbm.at[pl.ds(lo + (NCHA - 2) * RA, RA), :], sout0).wait()
    pltpu.make_async_copy(
        bout1, out_hbm.at[pl.ds(lo + (NCHA - 1) * RA, RA), :], sout1).wait()


def kernel(data, batch_id, batch_size, weights, bias):
    del batch_size
    ids = batch_id.astype(jnp.int32)
    mesh = plsc.VectorSubcoreMesh(core_axis_name="c", subcore_axis_name="s",
                                  num_cores=NC, num_subcores=NS)

    stats = pl.kernel(
        _stats_body,
        out_type=(jax.ShapeDtypeStruct((NW, SD), jnp.float32),
                  jax.ShapeDtypeStruct((NW, B), jnp.int32)),
        mesh=mesh,
        scratch_types=[
            pltpu.VMEM((PER,), jnp.int32),
            pltpu.VMEM((SD,), jnp.float32),
            pltpu.VMEM((2 * B,), jnp.int32),
            pltpu.VMEM((B,), jnp.int32),
            pltpu.VMEM((RS, C), jnp.float32),
            pltpu.VMEM((RS, C), jnp.float32),
            pltpu.SemaphoreType.DMA,
            pltpu.SemaphoreType.DMA,
        ],
        compiler_params=pltpu.CompilerParams(needs_layout_passes=False),
        name="octree_in_stats",
    )
    part, cnt = stats(data, ids)

    apply_k = pl.kernel(
        _apply_body,
        out_type=jax.ShapeDtypeStruct((N, C), jnp.float32),
        mesh=mesh,
        scratch_types=[
            pltpu.VMEM((2 * B,), jnp.int32),
            pltpu.VMEM((2 * B,), jnp.float32),
            pltpu.VMEM((NW, 2 * C), jnp.float32),
            pltpu.VMEM((NW, B), jnp.int32),
            pltpu.VMEM((1, C), jnp.float32),
            pltpu.VMEM((1, C), jnp.float32),
            pltpu.VMEM((B * C,), jnp.float32),
            pltpu.VMEM((B * C,), jnp.float32),
            pltpu.VMEM((RA, C), jnp.float32),
            pltpu.VMEM((RA, C), jnp.float32),
            pltpu.VMEM((RA, C), jnp.float32),
            pltpu.VMEM((RA, C), jnp.float32),
            pltpu.SemaphoreType.DMA,
            pltpu.SemaphoreType.DMA,
            pltpu.SemaphoreType.DMA,
            pltpu.SemaphoreType.DMA,
        ],
        compiler_params=pltpu.CompilerParams(needs_layout_passes=False),
        name="octree_in_apply",
    )
    return apply_k(data, part, cnt, weights, bias)


# speedup vs baseline: 1.6392x; 1.0461x over previous
import jax
import jax.numpy as jnp
from jax import lax
from jax.experimental import pallas as pl
from jax.experimental.pallas import tpu as pltpu
from jax.experimental.pallas import tpu_sc as plsc

N, C, B = 320000, 128, 16
NC, NS, L = 2, 16, 16
NW = NC * NS
PER = N // NW
RS = 400
NCHS = PER // RS
RA = 200
NCHA = PER // RA
CV = C // L
EPS = 1e-5
SD = B * 2 * C


def _worker_id():
    return lax.axis_index("s") * NC + lax.axis_index("c")


def _local_starts(ids_hbm, ids_v, cnt_v, starts_v, lo):
    pltpu.sync_copy(ids_hbm.at[pl.ds(lo, PER)], ids_v)
    cnt_v[...] = jnp.zeros((B,), jnp.int32)
    ones = jnp.ones((L,), jnp.int32)

    def _hist(i, carry):
        v = ids_v[pl.ds(i * L, L)]
        plsc.addupdate_scatter(cnt_v, [v], ones)
        return carry

    lax.fori_loop(0, PER // L, _hist, 0)
    cnts = cnt_v[...]
    st16 = plsc.cumsum(cnts) - cnts
    starts_v[pl.ds(0, L)] = st16
    starts_v[pl.ds(L, L)] = jnp.full((L,), PER, jnp.int32)
    return st16


def _seg_bounds(starts_v, b, g_lo, g_hi):
    sv = starts_v[pl.ds(b, L)]
    a = jnp.maximum(sv[0], g_lo)
    e = jnp.minimum(sv[1], g_hi)
    return a, e


def _stats_body(data_hbm, ids_hbm, part_hbm, cnt_hbm,
                ids_v, acc_v, starts_v, cnt_v, buf0, buf1, sin0, sin1):
    wid = _worker_id()
    lo = wid * PER

    pltpu.async_copy(data_hbm.at[pl.ds(lo, RS), :], buf0, sin0)
    pltpu.async_copy(data_hbm.at[pl.ds(lo + RS, RS), :], buf1, sin1)

    st16 = _local_starts(ids_hbm, ids_v, cnt_v, starts_v, lo)

    z = jnp.zeros((L,), jnp.float32)

    def _zero(i, carry):
        acc_v[pl.ds(i * L, L)] = z
        return carry

    lax.fori_loop(0, SD // L, _zero, 0)

    def _chunk(g, buf):
        g_lo = g * RS
        b_first = jnp.sum((st16 <= g_lo).astype(jnp.int32)) - 1
        b_last = jnp.sum((st16 <= g_lo + RS - 1).astype(jnp.int32)) - 1

        def _seg(b, carry):
            a, e = _seg_bounds(starts_v, b, g_lo, g_lo + RS)

            @pl.when(e > a)
            def _():
                def _rows(r, rcarry):
                    rr = r - g_lo
                    x = [buf[rr, pl.ds(j * L, L)] for j in range(CV)]
                    return (tuple(rcarry[j] + x[j] for j in range(CV))
                            + tuple(rcarry[CV + j] + x[j] * x[j]
                                    for j in range(CV)))

                init = tuple(jnp.zeros((L,), jnp.float32)
                             for _ in range(2 * CV))
                acc = lax.fori_loop(a, e, _rows, init)
                for j in range(CV):
                    plsc.addupdate(
                        acc_v.at[pl.ds(b * 2 * C + j * L, L)], acc[j])
                    plsc.addupdate(
                        acc_v.at[pl.ds(b * 2 * C + C + j * L, L)],
                        acc[CV + j])

            return carry

        lax.fori_loop(b_first, b_last + 1, _seg, 0)

    def _pair(p, carry):
        g0 = 2 * p
        pltpu.make_async_copy(
            data_hbm.at[pl.ds(lo + g0 * RS, RS), :], buf0, sin0).wait()
        _chunk(g0, buf0)

        @pl.when(g0 + 2 < NCHS)
        def _():
            pltpu.async_copy(
                data_hbm.at[pl.ds(lo + (g0 + 2) * RS, RS), :], buf0, sin0)

        pltpu.make_async_copy(
            data_hbm.at[pl.ds(lo + (g0 + 1) * RS, RS), :], buf1, sin1).wait()
        _chunk(g0 + 1, buf1)

        @pl.when(g0 + 3 < NCHS)
        def _():
            pltpu.async_copy(
                data_hbm.at[pl.ds(lo + (g0 + 3) * RS, RS), :], buf1, sin1)

        return carry

    lax.fori_loop(0, NCHS // 2, _pair, 0)

    pltpu.make_async_copy(
        data_hbm.at[pl.ds(lo + (NCHS - 1) * RS, RS), :], buf0, sin0).wait()
    _chunk(NCHS - 1, buf0)

    pltpu.sync_copy(acc_v, part_hbm.at[wid])
    pltpu.sync_copy(cnt_v, cnt_hbm.at[wid])


def _apply_body(data_hbm, part_hbm, cnt_hbm, w_hbm, bias_hbm,
                out_hbm,
                starts_v, stat_v, pbuf, cbuf, w_v, bias_v,
                sc_v, sh_v, bin0, bin1, bout0, bout1,
                sin0, sin1, sout0, sout1):
    wid = _worker_id()
    lo = wid * PER

    pltpu.async_copy(data_hbm.at[pl.ds(lo, RA), :], bin0, sin0)
    pltpu.async_copy(data_hbm.at[pl.ds(lo + RA, RA), :], bin1, sin1)

    pltpu.sync_copy(cnt_hbm, cbuf)
    tot = jnp.zeros((B,), jnp.int32)
    for w2 in range(NW):
        tot = tot + cbuf[w2, :]
    own = cbuf[wid, :]
    st16 = plsc.cumsum(own) - own
    starts_v[pl.ds(0, L)] = st16
    starts_v[pl.ds(L, L)] = jnp.full((L,), PER, jnp.int32)

    pltpu.sync_copy(w_hbm, w_v)
    pltpu.sync_copy(bias_hbm, bias_v)

    totf = tot.astype(jnp.float32)
    stat_v[pl.ds(0, L)] = totf
    stat_v[pl.ds(L, L)] = jnp.full((L,), 1.0, jnp.float32)

    b_lo = jnp.sum((st16 <= 0).astype(jnp.int32)) - 1
    b_hi = jnp.sum((st16 <= PER - 1).astype(jnp.int32)) - 1

    def _mkseg(b, carry):
        pltpu.sync_copy(part_hbm.at[:, pl.ds(b * 2 * C, 2 * C)], pbuf)

        def _racc(w2, acc):
            return tuple(acc[j] + pbuf[w2, pl.ds(j * L, L)]
                         for j in range(2 * CV))

        red = lax.fori_loop(
            0, NW, _racc,
            tuple(jnp.zeros((L,), jnp.float32) for _ in range(2 * CV)))

        n = jnp.full((L,), stat_v[pl.ds(b, L)][0])
        eta = 1.0 / (n + EPS)
        for j in range(CV):
            s_sum = red[j]
            q_sum = red[CV + j]
            mean = s_sum * eta
            var = eta * q_sum - (eta * eta) * (s_sum * s_sum) * (2.0 - n * eta)
            v = var + EPS
            i = plsc.bitcast(v, jnp.int32)
            y = plsc.bitcast(jnp.int32(0x5F3759DF) - (i >> 1), jnp.float32)
            for _ in range(3):
                y = y * (1.5 - 0.5 * v * y * y)
            wv = w_v[0, pl.ds(j * L, L)]
            bv = bias_v[0, pl.ds(j * L, L)]
            scale = y * wv
            sc_v[pl.ds(b * C + j * L, L)] = scale
            sh_v[pl.ds(b * C + j * L, L)] = bv - mean * scale
        return carry

    lax.fori_loop(b_lo, b_hi + 1, _mkseg, 0)

    def _chunk(g, bin_, bout):
        g_lo = g * RA
        b_first = jnp.sum((st16 <= g_lo).astype(jnp.int32)) - 1
        b_last = jnp.sum((st16 <= g_lo + RA - 1).astype(jnp.int32)) - 1

        def _seg(b, carry):
            a, e = _seg_bounds(starts_v, b, g_lo, g_lo + RA)

            @pl.when(e > a)
            def _():
                sc = [sc_v[pl.ds(b * C + j * L, L)] for j in range(CV)]
                sh = [sh_v[pl.ds(b * C + j * L, L)] for j in range(CV)]

                @plsc.parallel_loop(a - g_lo, e - g_lo)
                def _rows(rr):
                    x = [bin_[rr, pl.ds(j * L, L)] for j in range(CV)]
                    for j in range(CV):
                        bout[rr, pl.ds(j * L, L)] = x[j] * sc[j] + sh[j]

            return carry

        lax.fori_loop(b_first, b_last + 1, _seg, 0)

    def _pair(p, carry):
        g0 = 2 * p

        @pl.when(g0 >= 2)
        def _():
            pltpu.make_async_copy(
                bout0, out_hbm.at[pl.ds(lo + (g0 - 2) * RA, RA), :],
                sout0).wait()

        pltpu.make_async_copy(
            data_hbm.at[pl.ds(lo + g0 * RA, RA), :], bin0, sin0).wait()
        _chunk(g0, bin0, bout0)
        pltpu.async_copy(
            bout0, out_hbm.at[pl.ds(lo + g0 * RA, RA), :], sout0)

        @pl.when(g0 + 2 < NCHA)
        def _():
            pltpu.async_copy(
                data_hbm.at[pl.ds(lo + (g0 + 2) * RA, RA), :], bin0, sin0)

        @pl.when(g0 >= 2)
        def _():
            pltpu.make_async_copy(
                bout1, out_hbm.at[pl.ds(lo + (g0 - 1) * RA, RA), :],
                sout1).wait()

        pltpu.make_async_copy(
            data_hbm.at[pl.ds(lo + (g0 + 1) * RA, RA), :], bin1, sin1).wait()
        _chunk(g0 + 1, bin1, bout1)
        pltpu.async_copy(
            bout1, out_hbm.at[pl.ds(lo + (g0 + 1) * RA, RA), :], sout1)

        @pl.when(g0 + 3 < NCHA)
        def _():
            pltpu.async_copy(
                data_hbm.at[pl.ds(lo + (g0 + 3) * RA, RA), :], bin1, sin1)

        return carry

    lax.fori_loop(0, NCHA // 2, _pair, 0)

    pltpu.make_async_copy(
        bout0, out_hbm.at[pl.ds(lo + (NCHA - 2) * RA, RA), :], sout0).wait()
    pltpu.make_async_copy(
        bout1, out_hbm.at[pl.ds(lo + (NCHA - 1) * RA, RA), :], sout1).wait()


def kernel(data, batch_id, batch_size, weights, bias):
    del batch_size
    ids = batch_id.astype(jnp.int32)
    mesh = plsc.VectorSubcoreMesh(core_axis_name="c", subcore_axis_name="s",
                                  num_cores=NC, num_subcores=NS)

    stats = pl.kernel(
        _stats_body,
        out_type=(jax.ShapeDtypeStruct((NW, SD), jnp.float32),
                  jax.ShapeDtypeStruct((NW, B), jnp.int32)),
        mesh=mesh,
        scratch_types=[
            pltpu.VMEM((PER,), jnp.int32),
            pltpu.VMEM((SD,), jnp.float32),
            pltpu.VMEM((2 * B,), jnp.int32),
            pltpu.VMEM((B,), jnp.int32),
            pltpu.VMEM((RS, C), jnp.float32),
            pltpu.VMEM((RS, C), jnp.float32),
            pltpu.SemaphoreType.DMA,
            pltpu.SemaphoreType.DMA,
        ],
        compiler_params=pltpu.CompilerParams(needs_layout_passes=False),
        name="octree_in_stats",
    )
    part, cnt = stats(data, ids)

    apply_k = pl.kernel(
        _apply_body,
        out_type=jax.ShapeDtypeStruct((N, C), jnp.float32),
        mesh=mesh,
        scratch_types=[
            pltpu.VMEM((2 * B,), jnp.int32),
            pltpu.VMEM((2 * B,), jnp.float32),
            pltpu.VMEM((NW, 2 * C), jnp.float32),
            pltpu.VMEM((NW, B), jnp.int32),
            pltpu.VMEM((1, C), jnp.float32),
            pltpu.VMEM((1, C), jnp.float32),
            pltpu.VMEM((B * C,), jnp.float32),
            pltpu.VMEM((B * C,), jnp.float32),
            pltpu.VMEM((RA, C), jnp.float32),
            pltpu.VMEM((RA, C), jnp.float32),
            pltpu.VMEM((RA, C), jnp.float32),
            pltpu.VMEM((RA, C), jnp.float32),
            pltpu.SemaphoreType.DMA,
            pltpu.SemaphoreType.DMA,
            pltpu.SemaphoreType.DMA,
            pltpu.SemaphoreType.DMA,
        ],
        compiler_params=pltpu.CompilerParams(needs_layout_passes=False),
        name="octree_in_apply",
    )
    return apply_k(data, part, cnt, weights, bias)


# speedup vs baseline: 1.6497x; 1.0065x over previous
import jax
import jax.numpy as jnp
from jax import lax
from jax.experimental import pallas as pl
from jax.experimental.pallas import tpu as pltpu
from jax.experimental.pallas import tpu_sc as plsc

N, C, B = 320000, 128, 16
NC, NS, L = 2, 16, 16
NW = NC * NS
PER = N // NW
RS = 400
NCHS = PER // RS
RA = 200
NCHA = PER // RA
CV = C // L
EPS = 1e-5
SD = B * 2 * C


def _worker_id():
    return lax.axis_index("s") * NC + lax.axis_index("c")


def _local_starts(ids_hbm, ids_v, cnt_v, starts_v, lo):
    pltpu.sync_copy(ids_hbm.at[pl.ds(lo, PER)], ids_v)
    cnt_v[...] = jnp.zeros((B,), jnp.int32)
    ones = jnp.ones((L,), jnp.int32)

    def _hist(i, carry):
        v = ids_v[pl.ds(i * L, L)]
        plsc.addupdate_scatter(cnt_v, [v], ones)
        return carry

    lax.fori_loop(0, PER // L, _hist, 0)
    cnts = cnt_v[...]
    st16 = plsc.cumsum(cnts) - cnts
    starts_v[pl.ds(0, L)] = st16
    starts_v[pl.ds(L, L)] = jnp.full((L,), PER, jnp.int32)
    return st16


def _seg_bounds(starts_v, b, g_lo, g_hi):
    sv = starts_v[pl.ds(b, L)]
    a = jnp.maximum(sv[0], g_lo)
    e = jnp.minimum(sv[1], g_hi)
    return a, e


def _stats_body(data_hbm, ids_hbm, part_hbm, cnt_hbm,
                ids_v, acc_v, starts_v, cnt_v, buf0, buf1, sin0, sin1):
    wid = _worker_id()
    lo = wid * PER

    pltpu.async_copy(data_hbm.at[pl.ds(lo, RS), :], buf0, sin0)
    pltpu.async_copy(data_hbm.at[pl.ds(lo + RS, RS), :], buf1, sin1)

    st16 = _local_starts(ids_hbm, ids_v, cnt_v, starts_v, lo)

    z = jnp.zeros((L,), jnp.float32)

    def _zero(i, carry):
        acc_v[pl.ds(i * L, L)] = z
        return carry

    lax.fori_loop(0, SD // L, _zero, 0)

    def _chunk(g, buf):
        g_lo = g * RS
        b_first = jnp.sum((st16 <= g_lo).astype(jnp.int32)) - 1
        b_last = jnp.sum((st16 <= g_lo + RS - 1).astype(jnp.int32)) - 1

        def _seg(b, carry):
            a, e = _seg_bounds(starts_v, b, g_lo, g_lo + RS)

            @pl.when(e > a)
            def _():
                def _rows(r, rcarry):
                    rr = r - g_lo
                    x = [buf[rr, pl.ds(j * L, L)] for j in range(CV)]
                    return (tuple(rcarry[j] + x[j] for j in range(CV))
                            + tuple(rcarry[CV + j] + x[j] * x[j]
                                    for j in range(CV)))

                init = tuple(jnp.zeros((L,), jnp.float32)
                             for _ in range(2 * CV))
                acc = lax.fori_loop(a, e, _rows, init)
                for j in range(CV):
                    plsc.addupdate(
                        acc_v.at[pl.ds(b * 2 * C + j * L, L)], acc[j])
                    plsc.addupdate(
                        acc_v.at[pl.ds(b * 2 * C + C + j * L, L)],
                        acc[CV + j])

            return carry

        lax.fori_loop(b_first, b_last + 1, _seg, 0)

    def _pair(p, carry):
        g0 = 2 * p
        pltpu.make_async_copy(
            data_hbm.at[pl.ds(lo + g0 * RS, RS), :], buf0, sin0).wait()
        _chunk(g0, buf0)

        @pl.when(g0 + 2 < NCHS)
        def _():
            pltpu.async_copy(
                data_hbm.at[pl.ds(lo + (g0 + 2) * RS, RS), :], buf0, sin0)

        pltpu.make_async_copy(
            data_hbm.at[pl.ds(lo + (g0 + 1) * RS, RS), :], buf1, sin1).wait()
        _chunk(g0 + 1, buf1)

        @pl.when(g0 + 3 < NCHS)
        def _():
            pltpu.async_copy(
                data_hbm.at[pl.ds(lo + (g0 + 3) * RS, RS), :], buf1, sin1)

        return carry

    lax.fori_loop(0, NCHS // 2, _pair, 0)

    pltpu.make_async_copy(
        data_hbm.at[pl.ds(lo + (NCHS - 1) * RS, RS), :], buf0, sin0).wait()
    _chunk(NCHS - 1, buf0)

    pltpu.sync_copy(acc_v, part_hbm.at[wid])
    pltpu.sync_copy(cnt_v, cnt_hbm.at[wid])


def _apply_body(data_hbm, part_hbm, cnt_hbm, wb_hbm,
                out_hbm,
                starts_v, stat_v, pbuf, cbuf, wb_v,
                sc_v, sh_v, bin0, bin1, bout0, bout1,
                sin0, sin1, sout0, sout1, saux):
    wid = _worker_id()
    lo = wid * PER

    pltpu.async_copy(data_hbm.at[pl.ds(lo, RA), :], bin0, sin0)
    pltpu.async_copy(data_hbm.at[pl.ds(lo + RA, RA), :], bin1, sin1)
    cp_cnt = pltpu.async_copy(cnt_hbm, cbuf, saux)
    cp_wb = pltpu.async_copy(wb_hbm, wb_v, saux)
    cp_cnt.wait()
    cp_wb.wait()

    tot = jnp.zeros((B,), jnp.int32)
    for w2 in range(NW):
        tot = tot + cbuf[w2, :]
    own = cbuf[wid, :]
    st16 = plsc.cumsum(own) - own
    starts_v[pl.ds(0, L)] = st16
    starts_v[pl.ds(L, L)] = jnp.full((L,), PER, jnp.int32)

    totf = tot.astype(jnp.float32)
    stat_v[pl.ds(0, L)] = totf
    stat_v[pl.ds(L, L)] = jnp.full((L,), 1.0, jnp.float32)

    b_lo = jnp.sum((st16 <= 0).astype(jnp.int32)) - 1
    b_hi = jnp.sum((st16 <= PER - 1).astype(jnp.int32)) - 1

    def _mkseg(b, carry):
        pltpu.sync_copy(part_hbm.at[:, pl.ds(b * 2 * C, 2 * C)], pbuf)

        def _racc(w2, acc):
            return tuple(acc[j] + pbuf[w2, pl.ds(j * L, L)]
                         for j in range(2 * CV))

        red = lax.fori_loop(
            0, NW, _racc,
            tuple(jnp.zeros((L,), jnp.float32) for _ in range(2 * CV)))

        n = jnp.full((L,), stat_v[pl.ds(b, L)][0])
        eta = 1.0 / (n + EPS)
        for j in range(CV):
            s_sum = red[j]
            q_sum = red[CV + j]
            mean = s_sum * eta
            var = eta * q_sum - (eta * eta) * (s_sum * s_sum) * (2.0 - n * eta)
            v = var + EPS
            i = plsc.bitcast(v, jnp.int32)
            y = plsc.bitcast(jnp.int32(0x5F3759DF) - (i >> 1), jnp.float32)
            for _ in range(3):
                y = y * (1.5 - 0.5 * v * y * y)
            wv = wb_v[0, pl.ds(j * L, L)]
            bv = wb_v[1, pl.ds(j * L, L)]
            scale = y * wv
            sc_v[pl.ds(b * C + j * L, L)] = scale
            sh_v[pl.ds(b * C + j * L, L)] = bv - mean * scale
        return carry

    lax.fori_loop(b_lo, b_hi + 1, _mkseg, 0)

    def _chunk(g, bin_, bout):
        g_lo = g * RA
        b_first = jnp.sum((st16 <= g_lo).astype(jnp.int32)) - 1
        b_last = jnp.sum((st16 <= g_lo + RA - 1).astype(jnp.int32)) - 1

        def _seg(b, carry):
            a, e = _seg_bounds(starts_v, b, g_lo, g_lo + RA)

            @pl.when(e > a)
            def _():
                sc = [sc_v[pl.ds(b * C + j * L, L)] for j in range(CV)]
                sh = [sh_v[pl.ds(b * C + j * L, L)] for j in range(CV)]

                @plsc.parallel_loop(a - g_lo, e - g_lo)
                def _rows(rr):
                    x = [bin_[rr, pl.ds(j * L, L)] for j in range(CV)]
                    for j in range(CV):
                        bout[rr, pl.ds(j * L, L)] = x[j] * sc[j] + sh[j]

            return carry

        lax.fori_loop(b_first, b_last + 1, _seg, 0)

    def _pair(p, carry):
        g0 = 2 * p

        @pl.when(g0 >= 2)
        def _():
            pltpu.make_async_copy(
                bout0, out_hbm.at[pl.ds(lo + (g0 - 2) * RA, RA), :],
                sout0).wait()

        pltpu.make_async_copy(
            data_hbm.at[pl.ds(lo + g0 * RA, RA), :], bin0, sin0).wait()
        _chunk(g0, bin0, bout0)
        pltpu.async_copy(
            bout0, out_hbm.at[pl.ds(lo + g0 * RA, RA), :], sout0)

        @pl.when(g0 + 2 < NCHA)
        def _():
            pltpu.async_copy(
                data_hbm.at[pl.ds(lo + (g0 + 2) * RA, RA), :], bin0, sin0)

        @pl.when(g0 >= 2)
        def _():
            pltpu.make_async_copy(
                bout1, out_hbm.at[pl.ds(lo + (g0 - 1) * RA, RA), :],
                sout1).wait()

        pltpu.make_async_copy(
            data_hbm.at[pl.ds(lo + (g0 + 1) * RA, RA), :], bin1, sin1).wait()
        _chunk(g0 + 1, bin1, bout1)
        pltpu.async_copy(
            bout1, out_hbm.at[pl.ds(lo + (g0 + 1) * RA, RA), :], sout1)

        @pl.when(g0 + 3 < NCHA)
        def _():
            pltpu.async_copy(
                data_hbm.at[pl.ds(lo + (g0 + 3) * RA, RA), :], bin1, sin1)

        return carry

    lax.fori_loop(0, NCHA // 2, _pair, 0)

    pltpu.make_async_copy(
        bout0, out_hbm.at[pl.ds(lo + (NCHA - 2) * RA, RA), :], sout0).wait()
    pltpu.make_async_copy(
        bout1, out_hbm.at[pl.ds(lo + (NCHA - 1) * RA, RA), :], sout1).wait()


def kernel(data, batch_id, batch_size, weights, bias):
    del batch_size
    ids = batch_id.astype(jnp.int32)
    mesh = plsc.VectorSubcoreMesh(core_axis_name="c", subcore_axis_name="s",
                                  num_cores=NC, num_subcores=NS)

    stats = pl.kernel(
        _stats_body,
        out_type=(jax.ShapeDtypeStruct((NW, SD), jnp.float32),
                  jax.ShapeDtypeStruct((NW, B), jnp.int32)),
        mesh=mesh,
        scratch_types=[
            pltpu.VMEM((PER,), jnp.int32),
            pltpu.VMEM((SD,), jnp.float32),
            pltpu.VMEM((2 * B,), jnp.int32),
            pltpu.VMEM((B,), jnp.int32),
            pltpu.VMEM((RS, C), jnp.float32),
            pltpu.VMEM((RS, C), jnp.float32),
            pltpu.SemaphoreType.DMA,
            pltpu.SemaphoreType.DMA,
        ],
        compiler_params=pltpu.CompilerParams(needs_layout_passes=False),
        name="octree_in_stats",
    )
    part, cnt = stats(data, ids)

    apply_k = pl.kernel(
        _apply_body,
        out_type=jax.ShapeDtypeStruct((N, C), jnp.float32),
        mesh=mesh,
        scratch_types=[
            pltpu.VMEM((2 * B,), jnp.int32),
            pltpu.VMEM((2 * B,), jnp.float32),
            pltpu.VMEM((NW, 2 * C), jnp.float32),
            pltpu.VMEM((NW, B), jnp.int32),
            pltpu.VMEM((2, C), jnp.float32),
            pltpu.VMEM((B * C,), jnp.float32),
            pltpu.VMEM((B * C,), jnp.float32),
            pltpu.VMEM((RA, C), jnp.float32),
            pltpu.VMEM((RA, C), jnp.float32),
            pltpu.VMEM((RA, C), jnp.float32),
            pltpu.VMEM((RA, C), jnp.float32),
            pltpu.SemaphoreType.DMA,
            pltpu.SemaphoreType.DMA,
            pltpu.SemaphoreType.DMA,
            pltpu.SemaphoreType.DMA,
            pltpu.SemaphoreType.DMA,
        ],
        compiler_params=pltpu.CompilerParams(needs_layout_passes=False),
        name="octree_in_apply",
    )
    wb = jnp.concatenate([weights, bias], axis=0)
    return apply_k(data, part, cnt, wb)
